# baseline (speedup 1.0000x reference)
import jax
import jax.numpy as jnp
from jax.experimental import pallas as pl
from jax.experimental.pallas import tpu as pltpu

_B, _T = 128, 1024
_CIN = 15
_EMB = 8
_CH = 64
_CTOT_PAD = 24
_DILS = (1, 2, 4, 8, 16)
_BB = 8
_NBLK = _BB * _T


def _tcn_kernel(xin_ref, w0a_ref, w0b_ref, w0r_ref, wa_ref, wb_ref,
                wh_ref, aux_ref, out_ref):
    n = _NBLK
    tia = jax.lax.broadcasted_iota(jnp.int32, (1, n), 1)
    tin = jnp.bitwise_and(tia, _T - 1)
    masks = {s: (tin >= s).astype(jnp.float32) for s in (1, 2, 4, 8, 16, 32)}

    def shift(v, s):
        rolled = jnp.concatenate([v[:, n - s:], v[:, :n - s]], axis=1)
        return rolled * masks[s]

    def conv3(v, wcat, bias, d):
        xcat = jnp.concatenate([shift(v, 2 * d), shift(v, d), v], axis=0)
        y = jnp.dot(wcat, xcat, preferred_element_type=jnp.float32)
        return y + bias

    aux = aux_ref[...]
    xin = xin_ref[...]

    h = jnp.maximum(conv3(xin, w0a_ref[...], aux[:, 0:1], 1), 0.0)
    h = jnp.maximum(conv3(h, w0b_ref[...], aux[:, 1:2], 1), 0.0)
    res = jnp.dot(w0r_ref[...], xin, preferred_element_type=jnp.float32)
    f = h + res + aux[:, 2:3]

    for i, d in enumerate(_DILS[1:]):
        h = jnp.maximum(conv3(f, wa_ref[i], aux[:, 3 + i:4 + i], d), 0.0)
        h = jnp.maximum(conv3(h, wb_ref[i], aux[:, 7 + i:8 + i], d), 0.0)
        f = f + h

    g = jnp.dot(wh_ref[...], f, preferred_element_type=jnp.float32)
    pet = jax.nn.softplus(g[0:1] + aux[0:1, 11:12])
    pck = jax.nn.softplus(g[1:2] + aux[0:1, 12:13])
    aet_lin = (g[2:3] + aux[0:1, 13:14]
               + aux[0:1, 14:15] * pet + aux[0:1, 15:16] * pck)
    aet = jax.nn.sigmoid(aet_lin) * pet
    out_ref[0:1, :] = pet
    out_ref[1:2, :] = pck
    out_ref[2:3, :] = aet
    out_ref[3:4, :] = pet - aet


def kernel(x, fveg_ids, fveg_emb, w0a, b0a, w0b, b0b, w0r, b0r,
           wa, ba, wb, bb, pet_w, pet_b, pck_w, pck_b, aet_w, aet_b):
    Bx, cin, Tt = x.shape
    nb = wa.shape[0]
    ntot = Bx * Tt

    fv = fveg_emb[fveg_ids]
    xin = jnp.concatenate(
        [x, jnp.broadcast_to(fv[:, :, None], (Bx, fv.shape[1], Tt)),
         jnp.zeros((Bx, _CTOT_PAD - cin - fv.shape[1], Tt), x.dtype)], axis=1)
    xin2 = xin.transpose(1, 0, 2).reshape(_CTOT_PAD, ntot)

    def cat_taps(w, ipad):
        w = jnp.pad(w, ((0, 0), (0, ipad - w.shape[1]), (0, 0)))
        return w.transpose(0, 2, 1).reshape(w.shape[0], 3 * ipad)

    w0a_c = cat_taps(w0a, _CTOT_PAD)
    w0b_c = cat_taps(w0b, _CH)
    w0r_c = jnp.pad(w0r[:, :, 0], ((0, 0), (0, _CTOT_PAD - cin - _EMB)))
    wa_c = wa.transpose(0, 1, 3, 2).reshape(nb, _CH, 3 * _CH)
    wb_c = wb.transpose(0, 1, 3, 2).reshape(nb, _CH, 3 * _CH)
    wh = jnp.concatenate([pet_w[:, :, 0], pck_w[:, :, 0],
                          aet_w[:, :_CH, 0],
                          jnp.zeros((5, _CH), x.dtype)], axis=0)

    aux = jnp.zeros((_CH, 16), jnp.float32)
    aux = aux.at[:, 0].set(b0a).at[:, 1].set(b0b).at[:, 2].set(b0r)
    aux = aux.at[:, 3:3 + nb].set(ba.T).at[:, 7:7 + nb].set(bb.T)
    aux = aux.at[0, 11].set(pet_b[0]).at[0, 12].set(pck_b[0])
    aux = aux.at[0, 13].set(aet_b[0])
    aux = aux.at[0, 14].set(aet_w[0, _CH, 0]).at[0, 15].set(aet_w[0, _CH + 1, 0])

    grid = (Bx // _BB,)
    out4 = pl.pallas_call(
        _tcn_kernel,
        grid=grid,
        in_specs=[
            pl.BlockSpec((_CTOT_PAD, _NBLK), lambda i: (0, i)),
            pl.BlockSpec((_CH, 3 * _CTOT_PAD), lambda i: (0, 0)),
            pl.BlockSpec((_CH, 3 * _CH), lambda i: (0, 0)),
            pl.BlockSpec((_CH, _CTOT_PAD), lambda i: (0, 0)),
            pl.BlockSpec((nb, _CH, 3 * _CH), lambda i: (0, 0, 0)),
            pl.BlockSpec((nb, _CH, 3 * _CH), lambda i: (0, 0, 0)),
            pl.BlockSpec((8, _CH), lambda i: (0, 0)),
            pl.BlockSpec((_CH, 16), lambda i: (0, 0)),
        ],
        out_specs=pl.BlockSpec((4, _NBLK), lambda i: (0, i)),
        out_shape=jax.ShapeDtypeStruct((4, ntot), jnp.float32),
        compiler_params=pltpu.CompilerParams(
            dimension_semantics=("parallel",),
            vmem_limit_bytes=56 * 1024 * 1024,
        ),
    )(xin2, w0a_c, w0b_c, w0r_c, wa_c, wb_c, wh, aux)

    o = out4.reshape(4, Bx, 1, Tt)
    return (o[0], o[1], o[2], o[3])
